# baseline (speedup 1.0000x reference)
import dataclasses
import functools

import jax
import jax.numpy as jnp
from jax import lax
from jax.experimental import pallas as pl
from jax.experimental.pallas import tpu as pltpu
from jax.experimental.pallas import tpu_sc as plsc

B = 16384
F = 26
V = 100000
D = 32
C = 16

NWORK = 32
ROWS_PER_WORKER = B // NWORK
WIN_ROWS = 8
NWIN = ROWS_PER_WORKER // WIN_ROWS
WLOOK = WIN_ROWS * F
NG = WLOOK // 16
HALF = WLOOK // 2

MLP_BLOCK = 1024


def _sc_gather(t2, cat_flat, off_c):
    mesh = plsc.VectorSubcoreMesh(core_axis_name="c", subcore_axis_name="s")
    cp = pltpu.CompilerParams(needs_layout_passes=False,
                              use_tc_tiling_on_sc=False)

    @functools.partial(
        pl.kernel,
        mesh=mesh,
        compiler_params=cp,
        out_type=jax.ShapeDtypeStruct((B * F, D), jnp.float32),
        scratch_types=[
            pltpu.VMEM((WLOOK,), jnp.int32),
            pltpu.VMEM((WLOOK,), jnp.int32),
            pltpu.VMEM((WLOOK,), jnp.int32),
            pltpu.VMEM((WLOOK, D), jnp.float32),
            pltpu.SemaphoreType.DMA,
        ],
    )
    def gather_kernel(t2_hbm, cat_hbm, off_hbm, out_hbm,
                      cat_v, off_v, tidx_v, rows_v, sem):
        wid = lax.axis_index("c") * 16 + lax.axis_index("s")
        pltpu.sync_copy(off_hbm, off_v)

        @pl.loop(0, NWIN)
        def _window(w):
            pos0 = (wid * ROWS_PER_WORKER + w * WIN_ROWS) * F
            pltpu.sync_copy(cat_hbm.at[pl.ds(pos0, WLOOK)], cat_v)

            @pl.loop(0, NG)
            def _idx(g):
                sl = pl.ds(g * 16, 16)
                tidx_v[sl] = cat_v[sl] + off_v[sl]

            cp1 = pltpu.async_copy(
                t2_hbm.at[tidx_v.at[pl.ds(0, HALF)]],
                rows_v.at[pl.ds(0, HALF)], sem)
            cp2 = pltpu.async_copy(
                t2_hbm.at[tidx_v.at[pl.ds(HALF, HALF)]],
                rows_v.at[pl.ds(HALF, HALF)], sem)
            cp1.wait()
            cp2.wait()

            pltpu.sync_copy(rows_v, out_hbm.at[pl.ds(pos0, WLOOK)])

    return gather_kernel(t2, cat_flat, off_c)


def _mlp_kernel(emb_ref, cont_ref, w0e_ref, w0c_ref, b0_ref, w1_ref, b1_ref,
                w2_ref, b2_ref, out_ref):
    x = jnp.dot(emb_ref[...], w0e_ref[...], preferred_element_type=jnp.float32)
    x = x + jnp.dot(cont_ref[...], w0c_ref[...], preferred_element_type=jnp.float32)
    x = jnp.maximum(x + b0_ref[...], 0.0)
    x = jnp.maximum(jnp.dot(x, w1_ref[...], preferred_element_type=jnp.float32)
                    + b1_ref[...], 0.0)
    x = jnp.maximum(jnp.dot(x, w2_ref[...], preferred_element_type=jnp.float32)
                    + b2_ref[...], 0.0)
    out_ref[...] = x


def _mlp(emb, cont, W0e, W0c, b0, W1, b1, W2, b2):
    grid = (B // MLP_BLOCK,)
    full = lambda shape: pl.BlockSpec(shape, lambda i: (0, 0))
    return pl.pallas_call(
        _mlp_kernel,
        grid=grid,
        in_specs=[
            pl.BlockSpec((MLP_BLOCK, F * D), lambda i: (i, 0)),
            pl.BlockSpec((MLP_BLOCK, C), lambda i: (i, 0)),
            full(W0e.shape), full(W0c.shape), full(b0.shape),
            full(W1.shape), full(b1.shape), full(W2.shape), full(b2.shape),
        ],
        out_specs=pl.BlockSpec((MLP_BLOCK, W2.shape[1]), lambda i: (i, 0)),
        out_shape=jax.ShapeDtypeStruct((B, W2.shape[1]), jnp.float32),
    )(emb, cont, W0e, W0c, b0, W1, b1, W2, b2)


def kernel(continuous, categorical_indices, tables, W0, b0, W1, b1, W2, b2):
    cat_flat = categorical_indices.reshape(B * F)
    j = jnp.arange(WLOOK, dtype=jnp.int32)
    off_c = (j % F) * V
    emb = _sc_gather(tables, cat_flat, off_c).reshape(B, F * D)
    W0e = W0[: F * D]
    W0c = W0[F * D:]
    return _mlp(emb, continuous, W0e, W0c, b0[None, :], W1, b1[None, :],
                W2, b2[None, :])
